# Initial kernel scaffold; baseline (speedup 1.0000x reference)
import jax
import jax.numpy as jnp
from jax.experimental import pallas as pl


def kernel(data, indices):
    raise NotImplementedError("write your pallas kernel here")


# speedup vs baseline: 1.2328x; 1.2328x over previous
import functools

import jax
import jax.numpy as jnp
from jax import lax
from jax.experimental import pallas as pl
from jax.experimental.pallas import tpu as pltpu
from jax.experimental.pallas import tpu_sc as plsc

B, N, D = 16, 50000, 128
M = 2048
NC, NS, L = 2, 16, 16
NW = NC * NS
RPW = (B * M) // NW
CHUNK = 128


@functools.partial(
    pl.kernel,
    mesh=plsc.VectorSubcoreMesh(core_axis_name="c", subcore_axis_name="s"),
    out_type=jax.ShapeDtypeStruct((B * M, D), jnp.float32),
    scratch_types=[
        pltpu.VMEM((RPW,), jnp.int32),
        pltpu.VMEM((CHUNK, D), jnp.float32),
        pltpu.SemaphoreType.DMA,
    ],
)
def _gather_sc(data_hbm, idx_hbm, out_hbm, idx_v, buf_v, sem):
    wid = lax.axis_index("s") * NC + lax.axis_index("c")
    base = wid * RPW
    batch = base // M
    off = batch * N

    pltpu.sync_copy(idx_hbm.at[pl.ds(base, RPW)], idx_v)
    for i in range(RPW // L):
        sl = pl.ds(i * L, L)
        idx_v[sl] = idx_v[sl] + off

    for j in range(RPW // CHUNK):
        pltpu.async_copy(
            data_hbm.at[idx_v.at[pl.ds(j * CHUNK, CHUNK)]], buf_v, sem
        ).wait()
        pltpu.sync_copy(buf_v, out_hbm.at[pl.ds(base + j * CHUNK, CHUNK)])


def kernel(data, indices):
    data_flat = data.reshape(B * N, D)
    idx_flat = indices.reshape(B * M).astype(jnp.int32)
    out = _gather_sc(data_flat, idx_flat)
    return out.reshape(B, M, D)


# speedup vs baseline: 1.4331x; 1.1625x over previous
import functools

import jax
import jax.numpy as jnp
from jax import lax
from jax.experimental import pallas as pl
from jax.experimental.pallas import tpu as pltpu
from jax.experimental.pallas import tpu_sc as plsc

B, N, D = 16, 50000, 128
M = 2048
NC, NS, L = 2, 16, 16
NW = NC * NS
RPW = (B * M) // NW
CHUNK = 128
NCH = RPW // CHUNK
NBUF = 4


@functools.partial(
    pl.kernel,
    mesh=plsc.VectorSubcoreMesh(core_axis_name="c", subcore_axis_name="s"),
    out_type=jax.ShapeDtypeStruct((B * M, D), jnp.float32),
    scratch_types=[
        pltpu.VMEM((RPW,), jnp.int32),
        pltpu.VMEM((NBUF, CHUNK, D), jnp.float32),
        *([pltpu.SemaphoreType.DMA] * (2 * NBUF)),
    ],
)
def _gather_sc(data_hbm, idx_hbm, out_hbm, idx_v, bufs, *sems):
    gsem, wsem = sems[:NBUF], sems[NBUF:]
    wid = lax.axis_index("s") * NC + lax.axis_index("c")
    base = wid * RPW
    batch = base // M
    off = batch * N

    pltpu.sync_copy(idx_hbm.at[pl.ds(base, RPW)], idx_v)
    for i in range(RPW // L):
        sl = pl.ds(i * L, L)
        idx_v[sl] = idx_v[sl] + off

    def start_gather(j):
        b = j % NBUF
        return pltpu.async_copy(
            data_hbm.at[idx_v.at[pl.ds(j * CHUNK, CHUNK)]], bufs.at[b], gsem[b]
        )

    gd, wd = {}, {}
    for j in range(NBUF):
        gd[j] = start_gather(j)
    for j in range(NCH):
        b = j % NBUF
        gd[j].wait()
        wd[j] = pltpu.async_copy(
            bufs.at[b], out_hbm.at[pl.ds(base + j * CHUNK, CHUNK)], wsem[b]
        )
        if j + NBUF < NCH:
            wd[j].wait()
            gd[j + NBUF] = start_gather(j + NBUF)
    for j in range(max(0, NCH - NBUF), NCH):
        wd[j].wait()


def kernel(data, indices):
    data_flat = data.reshape(B * N, D)
    idx_flat = indices.reshape(B * M).astype(jnp.int32)
    out = _gather_sc(data_flat, idx_flat)
    return out.reshape(B, M, D)


# speedup vs baseline: 1.4903x; 1.0399x over previous
import functools

import jax
import jax.numpy as jnp
from jax import lax
from jax.experimental import pallas as pl
from jax.experimental.pallas import tpu as pltpu
from jax.experimental.pallas import tpu_sc as plsc

B, N, D = 16, 50000, 128
M = 2048
NC, NS, L = 2, 16, 16
NW = NC * NS
RPW = (B * M) // NW
CHUNK = 128
NCH = RPW // CHUNK
NBUF = 6


@functools.partial(
    pl.kernel,
    mesh=plsc.VectorSubcoreMesh(core_axis_name="c", subcore_axis_name="s"),
    out_type=jax.ShapeDtypeStruct((B * M, D), jnp.float32),
    scratch_types=[
        pltpu.VMEM((RPW,), jnp.int32),
        pltpu.VMEM((NBUF, CHUNK, D), jnp.float32),
        *([pltpu.SemaphoreType.DMA] * (2 * NBUF)),
    ],
)
def _gather_sc(data_hbm, idx_hbm, out_hbm, idx_v, bufs, *sems):
    gsem, wsem = sems[:NBUF], sems[NBUF:]
    wid = lax.axis_index("s") * NC + lax.axis_index("c")
    base = wid * RPW
    batch = base // M
    off = batch * N

    pltpu.sync_copy(idx_hbm.at[pl.ds(base, RPW)], idx_v)
    for i in range(RPW // L):
        sl = pl.ds(i * L, L)
        idx_v[sl] = idx_v[sl] + off

    def start_gather(j):
        b = j % NBUF
        return pltpu.async_copy(
            data_hbm.at[idx_v.at[pl.ds(j * CHUNK, CHUNK)]], bufs.at[b], gsem[b]
        )

    gd, wd = {}, {}
    for j in range(NBUF):
        gd[j] = start_gather(j)
    for j in range(NCH):
        b = j % NBUF
        gd[j].wait()
        wd[j] = pltpu.async_copy(
            bufs.at[b], out_hbm.at[pl.ds(base + j * CHUNK, CHUNK)], wsem[b]
        )
        if j + NBUF < NCH:
            wd[j].wait()
            gd[j + NBUF] = start_gather(j + NBUF)
    for j in range(max(0, NCH - NBUF), NCH):
        wd[j].wait()


def kernel(data, indices):
    data_flat = data.reshape(B * N, D)
    idx_flat = indices.reshape(B * M).astype(jnp.int32)
    out = _gather_sc(data_flat, idx_flat)
    return out.reshape(B, M, D)


# speedup vs baseline: 1.4998x; 1.0063x over previous
import functools

import jax
import jax.numpy as jnp
from jax import lax
from jax.experimental import pallas as pl
from jax.experimental.pallas import tpu as pltpu
from jax.experimental.pallas import tpu_sc as plsc

B, N, D = 16, 50000, 128
M = 2048
NC, NS, L = 2, 16, 16
NW = NC * NS
RPW = (B * M) // NW
CHUNK = 128
NCH = RPW // CHUNK
NBUF = 6


@functools.partial(
    pl.kernel,
    mesh=plsc.VectorSubcoreMesh(core_axis_name="c", subcore_axis_name="s"),
    out_type=jax.ShapeDtypeStruct((B * M, D), jnp.float32),
    scratch_types=[
        pltpu.VMEM((RPW,), jnp.int32),
        pltpu.VMEM((NBUF, CHUNK, D), jnp.float32),
        *([pltpu.SemaphoreType.DMA] * (2 * NBUF)),
    ],
)
def _gather_sc(data_hbm, idx_hbm, out_hbm, idx_v, bufs, *sems):
    gsem, wsem = sems[:NBUF], sems[NBUF:]
    wid = lax.axis_index("s") * NC + lax.axis_index("c")
    base = wid * RPW
    batch = base // M
    half = base % M
    off = batch * N

    pltpu.sync_copy(idx_hbm.at[batch, pl.ds(half, RPW)], idx_v)
    for i in range(RPW // L):
        sl = pl.ds(i * L, L)
        idx_v[sl] = idx_v[sl] + off

    def start_gather(j):
        b = j % NBUF
        return pltpu.async_copy(
            data_hbm.at[idx_v.at[pl.ds(j * CHUNK, CHUNK)]], bufs.at[b], gsem[b]
        )

    gd, wd = {}, {}
    for j in range(NBUF):
        gd[j] = start_gather(j)
    for j in range(NCH):
        b = j % NBUF
        gd[j].wait()
        wd[j] = pltpu.async_copy(
            bufs.at[b], out_hbm.at[pl.ds(base + j * CHUNK, CHUNK)], wsem[b]
        )
        if j + NBUF < NCH:
            wd[j].wait()
            gd[j + NBUF] = start_gather(j + NBUF)
    for j in range(max(0, NCH - NBUF), NCH):
        wd[j].wait()


def kernel(data, indices):
    data_flat = data.reshape(B * N, D)
    out = _gather_sc(data_flat, indices.astype(jnp.int32))
    return out.reshape(B, M, D)
